# baseline (speedup 1.0000x reference)
import jax
import jax.numpy as jnp
from jax.experimental import pallas as pl
from jax.experimental.pallas import tpu as pltpu

_G = 512
_BR = 1280
_G8 = _BR // 8
_K = 128


def _pass_a(lo_ref, hi_ref, x_ref, bc_ref, w1_ref, w2_ref, y_ref,
            sum_ref, max_ref):
    i = pl.program_id(0)

    @pl.when(i == 0)
    def _init():
        sum_ref[...] = jnp.zeros_like(sum_ref)
        max_ref[...] = jnp.full_like(max_ref, -jnp.inf)

    b = bc_ref[0, :, :]
    x = x_ref[...]
    s_lo = lo_ref[i]
    s_hi = hi_ref[i]

    bg = b.reshape(_G8, 8, 1)
    gfirst = bg[:, 0:1, :]
    glast = bg[:, 7:8, :]
    mfirst = bg == gfirst
    mlast = bg == glast
    has_mid = jnp.any(jnp.logical_not(jnp.logical_or(mfirst, mlast)))

    @pl.when(jnp.logical_not(has_mid))
    def _fast():
        xg = x.reshape(_G8, 8, 128)
        sumf = jnp.sum(jnp.where(mfirst, xg, 0.0), axis=1)
        maxf = jnp.max(jnp.where(mfirst, xg, -jnp.inf), axis=1)
        notf = jnp.logical_not(mfirst)
        suml = jnp.sum(jnp.where(notf, xg, 0.0), axis=1)
        maxl = jnp.max(jnp.where(notf, xg, -jnp.inf), axis=1)
        gf = gfirst[:, 0, :]
        gl = glast[:, 0, :]

        def body(s, carry):
            mf = gf == s
            ml = jnp.logical_and(gl == s, gf != s)
            ssum = (jnp.sum(jnp.where(mf, sumf, 0.0), axis=0, keepdims=True)
                    + jnp.sum(jnp.where(ml, suml, 0.0), axis=0, keepdims=True))
            smax = jnp.maximum(
                jnp.max(jnp.where(mf, maxf, -jnp.inf), axis=0, keepdims=True),
                jnp.max(jnp.where(ml, maxl, -jnp.inf), axis=0, keepdims=True))
            sum_ref[pl.ds(s, 1), :] = sum_ref[pl.ds(s, 1), :] + ssum
            max_ref[pl.ds(s, 1), :] = jnp.maximum(max_ref[pl.ds(s, 1), :], smax)
            return carry

        jax.lax.fori_loop(s_lo, s_hi + 1, body, 0)

    @pl.when(has_mid)
    def _slow():
        def body(s, carry):
            m = b == s
            mx = jnp.max(jnp.where(m, x, -jnp.inf), axis=0, keepdims=True)
            sm = jnp.sum(jnp.where(m, x, 0.0), axis=0, keepdims=True)
            sum_ref[pl.ds(s, 1), :] = sum_ref[pl.ds(s, 1), :] + sm
            max_ref[pl.ds(s, 1), :] = jnp.maximum(max_ref[pl.ds(s, 1), :], mx)
            return carry

        jax.lax.fori_loop(s_lo, s_hi + 1, body, 0)

    @pl.when(i == pl.num_programs(0) - 1)
    def _finish():
        mx = max_ref[...]
        mx = jnp.where(mx == -jnp.inf, 0.0, mx)
        sm = sum_ref[...]
        w1 = w1_ref[...]
        w2 = w2_ref[...]
        h1 = jnp.maximum(jnp.dot(mx, w1, preferred_element_type=jnp.float32), 0.0)
        o1 = jnp.dot(h1, w2, preferred_element_type=jnp.float32)
        h2 = jnp.maximum(jnp.dot(sm, w1, preferred_element_type=jnp.float32), 0.0)
        o2 = jnp.dot(h2, w2, preferred_element_type=jnp.float32)
        y_ref[...] = jnp.maximum(o1 + o2, 0.0)


def _pass_b(lo_ref, hi_ref, x_ref, bc_ref, y_ref, o_ref):
    i = pl.program_id(0)
    b = bc_ref[0, :, :]
    s_lo = lo_ref[i]
    s_hi = hi_ref[i]

    wlo = (s_lo // 8) * 8
    kio = jax.lax.broadcasted_iota(jnp.int32, (_BR, _K), 1)
    m = (kio == (b - wlo)).astype(jnp.float32)
    ys = y_ref[pl.ds(wlo, _K), :]
    rows = jax.lax.dot_general(m, ys, (((1,), (0,)), ((), ())),
                               preferred_element_type=jnp.float32)
    o_ref[...] = rows

    def body(s, carry):
        yy = y_ref[pl.ds(s, 1), :]
        mm = b == s
        o_ref[...] = jnp.where(mm, yy, o_ref[...])
        return carry

    jax.lax.fori_loop(wlo + _K, s_hi + 1, body, 0)
    o_ref[...] = o_ref[...] * x_ref[...]


def kernel(x, batch, W1, W2):
    n, c = x.shape
    nb = n // _BR
    bcol = batch.reshape(nb, _BR, 1)
    blo = bcol[:, 0, 0]
    bhi = bcol[:, _BR - 1, 0]

    y = pl.pallas_call(
        _pass_a,
        grid=(nb,),
        in_specs=[
            pl.BlockSpec(memory_space=pltpu.SMEM),
            pl.BlockSpec(memory_space=pltpu.SMEM),
            pl.BlockSpec((_BR, c), lambda i: (i, 0)),
            pl.BlockSpec((1, _BR, 1), lambda i: (i, 0, 0)),
            pl.BlockSpec((c, c // 8), lambda i: (0, 0)),
            pl.BlockSpec((c // 8, c), lambda i: (0, 0)),
        ],
        out_specs=pl.BlockSpec((_G, c), lambda i: (0, 0)),
        out_shape=jax.ShapeDtypeStruct((_G, c), jnp.float32),
        scratch_shapes=[
            pltpu.VMEM((_G, c), jnp.float32),
            pltpu.VMEM((_G, c), jnp.float32),
        ],
        compiler_params=pltpu.CompilerParams(
            dimension_semantics=("arbitrary",),
        ),
    )(blo, bhi, x, bcol, W1, W2)

    ypad = jnp.concatenate([y, jnp.zeros((_K, c), jnp.float32)], axis=0)

    out = pl.pallas_call(
        _pass_b,
        grid=(nb,),
        in_specs=[
            pl.BlockSpec(memory_space=pltpu.SMEM),
            pl.BlockSpec(memory_space=pltpu.SMEM),
            pl.BlockSpec((_BR, c), lambda i: (i, 0)),
            pl.BlockSpec((1, _BR, 1), lambda i: (i, 0, 0)),
            pl.BlockSpec((_G + _K, c), lambda i: (0, 0)),
        ],
        out_specs=pl.BlockSpec((_BR, c), lambda i: (i, 0)),
        out_shape=jax.ShapeDtypeStruct((n, c), jnp.float32),
        compiler_params=pltpu.CompilerParams(
            dimension_semantics=("arbitrary",),
        ),
    )(blo, bhi, x, bcol, ypad)
    return out


# speedup vs baseline: 1.6756x; 1.6756x over previous
import jax
import jax.numpy as jnp
from jax.experimental import pallas as pl
from jax.experimental.pallas import tpu as pltpu

_G = 512
_BR = 1280
_K = 128


def _pass_a(lo_ref, hi_ref, x_ref, bc_ref, w1_ref, w2_ref, y_ref,
            sum_ref, max_ref, bbc_ref):
    i = pl.program_id(0)

    @pl.when(i == 0)
    def _init():
        sum_ref[...] = jnp.zeros_like(sum_ref)
        max_ref[...] = jnp.full_like(max_ref, -jnp.inf)

    b = bc_ref[0, :, :]
    s_lo = lo_ref[i]
    s_hi = hi_ref[i]

    bbc_ref[...] = jnp.broadcast_to(b, (_BR, 128))

    def body(s, carry):
        m = bbc_ref[...] == s
        x = x_ref[...]
        sm = jnp.sum(jnp.where(m, x, 0.0), axis=0, keepdims=True)
        mx = jnp.max(jnp.where(m, x, -jnp.inf), axis=0, keepdims=True)
        sum_ref[pl.ds(s, 1), :] = sum_ref[pl.ds(s, 1), :] + sm
        max_ref[pl.ds(s, 1), :] = jnp.maximum(max_ref[pl.ds(s, 1), :], mx)
        return carry

    jax.lax.fori_loop(s_lo, s_hi + 1, body, 0)

    @pl.when(i == pl.num_programs(0) - 1)
    def _finish():
        mx = max_ref[...]
        mx = jnp.where(mx == -jnp.inf, 0.0, mx)
        sm = sum_ref[...]
        w1 = w1_ref[...]
        w2 = w2_ref[...]
        h1 = jnp.maximum(jnp.dot(mx, w1, preferred_element_type=jnp.float32), 0.0)
        o1 = jnp.dot(h1, w2, preferred_element_type=jnp.float32)
        h2 = jnp.maximum(jnp.dot(sm, w1, preferred_element_type=jnp.float32), 0.0)
        o2 = jnp.dot(h2, w2, preferred_element_type=jnp.float32)
        y_ref[...] = jnp.maximum(o1 + o2, 0.0)


def _pass_b(lo_ref, hi_ref, x_ref, bc_ref, y_ref, o_ref):
    i = pl.program_id(0)
    b = bc_ref[0, :, :]
    s_lo = lo_ref[i]
    s_hi = hi_ref[i]

    wlo = (s_lo // 8) * 8
    kio = jax.lax.broadcasted_iota(jnp.int32, (_BR, _K), 1)
    m = (kio == (b - wlo)).astype(jnp.float32)
    ys = y_ref[pl.ds(wlo, _K), :]
    rows = jax.lax.dot_general(m, ys, (((1,), (0,)), ((), ())),
                               preferred_element_type=jnp.float32)
    o_ref[...] = rows

    def body(s, carry):
        yy = y_ref[pl.ds(s, 1), :]
        mm = b == s
        o_ref[...] = jnp.where(mm, yy, o_ref[...])
        return carry

    jax.lax.fori_loop(wlo + _K, s_hi + 1, body, 0)
    o_ref[...] = o_ref[...] * x_ref[...]


def kernel(x, batch, W1, W2):
    n, c = x.shape
    nb = n // _BR
    bcol = batch.reshape(nb, _BR, 1)
    blo = bcol[:, 0, 0]
    bhi = bcol[:, _BR - 1, 0]

    y = pl.pallas_call(
        _pass_a,
        grid=(nb,),
        in_specs=[
            pl.BlockSpec(memory_space=pltpu.SMEM),
            pl.BlockSpec(memory_space=pltpu.SMEM),
            pl.BlockSpec((_BR, c), lambda i: (i, 0)),
            pl.BlockSpec((1, _BR, 1), lambda i: (i, 0, 0)),
            pl.BlockSpec((c, c // 8), lambda i: (0, 0)),
            pl.BlockSpec((c // 8, c), lambda i: (0, 0)),
        ],
        out_specs=pl.BlockSpec((_G, c), lambda i: (0, 0)),
        out_shape=jax.ShapeDtypeStruct((_G, c), jnp.float32),
        scratch_shapes=[
            pltpu.VMEM((_G, c), jnp.float32),
            pltpu.VMEM((_G, c), jnp.float32),
            pltpu.VMEM((_BR, c), jnp.int32),
        ],
        compiler_params=pltpu.CompilerParams(
            dimension_semantics=("arbitrary",),
        ),
    )(blo, bhi, x, bcol, W1, W2)

    ypad = jnp.concatenate([y, jnp.zeros((_K, c), jnp.float32)], axis=0)

    out = pl.pallas_call(
        _pass_b,
        grid=(nb,),
        in_specs=[
            pl.BlockSpec(memory_space=pltpu.SMEM),
            pl.BlockSpec(memory_space=pltpu.SMEM),
            pl.BlockSpec((_BR, c), lambda i: (i, 0)),
            pl.BlockSpec((1, _BR, 1), lambda i: (i, 0, 0)),
            pl.BlockSpec((_G + _K, c), lambda i: (0, 0)),
        ],
        out_specs=pl.BlockSpec((_BR, c), lambda i: (i, 0)),
        out_shape=jax.ShapeDtypeStruct((n, c), jnp.float32),
        compiler_params=pltpu.CompilerParams(
            dimension_semantics=("arbitrary",),
        ),
    )(blo, bhi, x, bcol, ypad)
    return out


# speedup vs baseline: 1.7152x; 1.0236x over previous
import jax
import jax.numpy as jnp
from jax.experimental import pallas as pl
from jax.experimental.pallas import tpu as pltpu

_G = 512
_BRA = 1280
_BRB = 3200
_K = 128


def _pass_a(lo_ref, hi_ref, x_ref, bc_ref, br_ref, w1_ref, w2_ref, y_ref,
            sum_ref, max_ref, bbc_ref):
    i = pl.program_id(0)

    @pl.when(i == 0)
    def _init():
        sum_ref[...] = jnp.zeros_like(sum_ref)
        max_ref[...] = jnp.full_like(max_ref, -jnp.inf)

    b = bc_ref[0, :, :]
    brow = br_ref[0, :, :]
    x = x_ref[...]
    s_lo = lo_ref[i]
    s_hi = hi_ref[i]
    wlo = (s_lo // 8) * 8

    kio = jax.lax.broadcasted_iota(jnp.int32, (_K, _BRA), 0)
    mt = (kio == (brow - wlo)).astype(jnp.bfloat16)
    xhi = x.astype(jnp.bfloat16)
    xlo = (x - xhi.astype(jnp.float32)).astype(jnp.bfloat16)
    dn = (((1,), (0,)), ((), ()))
    part = (jax.lax.dot_general(mt, xhi, dn, preferred_element_type=jnp.float32)
            + jax.lax.dot_general(mt, xlo, dn, preferred_element_type=jnp.float32))
    sum_ref[pl.ds(wlo, _K), :] = sum_ref[pl.ds(wlo, _K), :] + part

    bbc_ref[...] = jnp.broadcast_to(b, (_BRA, 128))

    def mbody(s, carry):
        m = bbc_ref[...] == s
        mx = jnp.max(jnp.where(m, x_ref[...], -jnp.inf), axis=0, keepdims=True)
        max_ref[pl.ds(s, 1), :] = jnp.maximum(max_ref[pl.ds(s, 1), :], mx)
        return carry

    jax.lax.fori_loop(s_lo, s_hi + 1, mbody, 0)

    def sbody(s, carry):
        m = bbc_ref[...] == s
        sm = jnp.sum(jnp.where(m, x_ref[...], 0.0), axis=0, keepdims=True)
        sum_ref[pl.ds(s, 1), :] = sum_ref[pl.ds(s, 1), :] + sm
        return carry

    jax.lax.fori_loop(wlo + _K, s_hi + 1, sbody, 0)

    @pl.when(i == pl.num_programs(0) - 1)
    def _finish():
        mx = max_ref[0:_G, :]
        mx = jnp.where(mx == -jnp.inf, 0.0, mx)
        sm = sum_ref[0:_G, :]
        w1 = w1_ref[...]
        w2 = w2_ref[...]
        h1 = jnp.maximum(jnp.dot(mx, w1, preferred_element_type=jnp.float32), 0.0)
        o1 = jnp.dot(h1, w2, preferred_element_type=jnp.float32)
        h2 = jnp.maximum(jnp.dot(sm, w1, preferred_element_type=jnp.float32), 0.0)
        o2 = jnp.dot(h2, w2, preferred_element_type=jnp.float32)
        y_ref[...] = jnp.maximum(o1 + o2, 0.0)


def _pass_b(lo_ref, hi_ref, x_ref, bc_ref, y_ref, o_ref):
    i = pl.program_id(0)
    b = bc_ref[0, :, :]
    s_lo = lo_ref[i]
    s_hi = hi_ref[i]

    wlo = (s_lo // 8) * 8
    kio = jax.lax.broadcasted_iota(jnp.int32, (_BRB, _K), 1)
    m = (kio == (b - wlo)).astype(jnp.float32)
    ys = y_ref[pl.ds(wlo, _K), :]
    rows = jax.lax.dot_general(m, ys, (((1,), (0,)), ((), ())),
                               preferred_element_type=jnp.float32)
    o_ref[...] = rows

    def body(s, carry):
        yy = y_ref[pl.ds(s, 1), :]
        mm = b == s
        o_ref[...] = jnp.where(mm, yy, o_ref[...])
        return carry

    jax.lax.fori_loop(wlo + _K, s_hi + 1, body, 0)
    o_ref[...] = o_ref[...] * x_ref[...]


def kernel(x, batch, W1, W2):
    n, c = x.shape
    nba = n // _BRA
    bcola = batch.reshape(nba, _BRA, 1)
    browa = batch.reshape(nba, 1, _BRA)
    bloa = bcola[:, 0, 0]
    bhia = bcola[:, _BRA - 1, 0]

    y = pl.pallas_call(
        _pass_a,
        grid=(nba,),
        in_specs=[
            pl.BlockSpec(memory_space=pltpu.SMEM),
            pl.BlockSpec(memory_space=pltpu.SMEM),
            pl.BlockSpec((_BRA, c), lambda i: (i, 0)),
            pl.BlockSpec((1, _BRA, 1), lambda i: (i, 0, 0)),
            pl.BlockSpec((1, 1, _BRA), lambda i: (i, 0, 0)),
            pl.BlockSpec((c, c // 8), lambda i: (0, 0)),
            pl.BlockSpec((c // 8, c), lambda i: (0, 0)),
        ],
        out_specs=pl.BlockSpec((_G, c), lambda i: (0, 0)),
        out_shape=jax.ShapeDtypeStruct((_G, c), jnp.float32),
        scratch_shapes=[
            pltpu.VMEM((_G + _K, c), jnp.float32),
            pltpu.VMEM((_G, c), jnp.float32),
            pltpu.VMEM((_BRA, c), jnp.int32),
        ],
        compiler_params=pltpu.CompilerParams(
            dimension_semantics=("arbitrary",),
        ),
    )(bloa, bhia, x, bcola, browa, W1, W2)

    ypad = jnp.concatenate([y, jnp.zeros((_K, c), jnp.float32)], axis=0)

    nbb = n // _BRB
    bcolb = batch.reshape(nbb, _BRB, 1)
    blob = bcolb[:, 0, 0]
    bhib = bcolb[:, _BRB - 1, 0]

    out = pl.pallas_call(
        _pass_b,
        grid=(nbb,),
        in_specs=[
            pl.BlockSpec(memory_space=pltpu.SMEM),
            pl.BlockSpec(memory_space=pltpu.SMEM),
            pl.BlockSpec((_BRB, c), lambda i: (i, 0)),
            pl.BlockSpec((1, _BRB, 1), lambda i: (i, 0, 0)),
            pl.BlockSpec((_G + _K, c), lambda i: (0, 0)),
        ],
        out_specs=pl.BlockSpec((_BRB, c), lambda i: (i, 0)),
        out_shape=jax.ShapeDtypeStruct((n, c), jnp.float32),
        compiler_params=pltpu.CompilerParams(
            dimension_semantics=("arbitrary",),
        ),
    )(blob, bhib, x, bcolb, ypad)
    return out


# speedup vs baseline: 1.7385x; 1.0136x over previous
import jax
import jax.numpy as jnp
from jax.experimental import pallas as pl
from jax.experimental.pallas import tpu as pltpu

_G = 512
_BRA = 1280
_BRB = 3200
_K = 128


def _pass_a(lo_ref, hi_ref, x_ref, bc_ref, br_ref, w1_ref, w2_ref, y_ref,
            sum_ref, max_ref, bbc_ref):
    i = pl.program_id(0)

    @pl.when(i == 0)
    def _init():
        sum_ref[...] = jnp.zeros_like(sum_ref)
        max_ref[...] = jnp.full_like(max_ref, -jnp.inf)

    b = bc_ref[0, :, :]
    brow = br_ref[0, :, :]
    x = x_ref[...]
    s_lo = lo_ref[i]
    s_hi = hi_ref[i]
    wlo = (s_lo // 8) * 8

    bbc_ref[...] = jnp.broadcast_to(b, (_BRA, 128))

    def body(s, carry):
        m = bbc_ref[...] == s
        xv = x_ref[...]
        sm = jnp.sum(jnp.where(m, xv, 0.0), axis=0, keepdims=True)
        mx = jnp.max(jnp.where(m, xv, -jnp.inf), axis=0, keepdims=True)
        sum_ref[pl.ds(s, 1), :] = sum_ref[pl.ds(s, 1), :] + sm
        max_ref[pl.ds(s, 1), :] = jnp.maximum(max_ref[pl.ds(s, 1), :], mx)
        return carry

    jax.lax.fori_loop(s_lo, s_hi + 1, body, 0)

    @pl.when(i == pl.num_programs(0) - 1)
    def _finish():
        mx = max_ref[0:_G, :]
        mx = jnp.where(mx == -jnp.inf, 0.0, mx)
        sm = sum_ref[0:_G, :]
        w1 = w1_ref[...]
        w2 = w2_ref[...]
        h1 = jnp.maximum(jnp.dot(mx, w1, preferred_element_type=jnp.float32), 0.0)
        o1 = jnp.dot(h1, w2, preferred_element_type=jnp.float32)
        h2 = jnp.maximum(jnp.dot(sm, w1, preferred_element_type=jnp.float32), 0.0)
        o2 = jnp.dot(h2, w2, preferred_element_type=jnp.float32)
        y_ref[...] = jnp.maximum(o1 + o2, 0.0)


def _pass_b(lo_ref, hi_ref, x_ref, bc_ref, y_ref, o_ref):
    i = pl.program_id(0)
    b = bc_ref[0, :, :]
    s_lo = lo_ref[i]
    s_hi = hi_ref[i]

    wlo = (s_lo // 8) * 8
    kio = jax.lax.broadcasted_iota(jnp.int32, (_BRB, _K), 1)
    m = (kio == (b - wlo)).astype(jnp.float32)
    ys = y_ref[pl.ds(wlo, _K), :]
    rows = jax.lax.dot_general(m, ys, (((1,), (0,)), ((), ())),
                               preferred_element_type=jnp.float32)
    o_ref[...] = rows

    def body(s, carry):
        yy = y_ref[pl.ds(s, 1), :]
        mm = b == s
        o_ref[...] = jnp.where(mm, yy, o_ref[...])
        return carry

    jax.lax.fori_loop(wlo + _K, s_hi + 1, body, 0)
    o_ref[...] = o_ref[...] * x_ref[...]


def kernel(x, batch, W1, W2):
    n, c = x.shape
    nba = n // _BRA
    bcola = batch.reshape(nba, _BRA, 1)
    browa = batch.reshape(nba, 1, _BRA)
    bloa = bcola[:, 0, 0]
    bhia = bcola[:, _BRA - 1, 0]

    y = pl.pallas_call(
        _pass_a,
        grid=(nba,),
        in_specs=[
            pl.BlockSpec(memory_space=pltpu.SMEM),
            pl.BlockSpec(memory_space=pltpu.SMEM),
            pl.BlockSpec((_BRA, c), lambda i: (i, 0)),
            pl.BlockSpec((1, _BRA, 1), lambda i: (i, 0, 0)),
            pl.BlockSpec((1, 1, _BRA), lambda i: (i, 0, 0)),
            pl.BlockSpec((c, c // 8), lambda i: (0, 0)),
            pl.BlockSpec((c // 8, c), lambda i: (0, 0)),
        ],
        out_specs=pl.BlockSpec((_G, c), lambda i: (0, 0)),
        out_shape=jax.ShapeDtypeStruct((_G, c), jnp.float32),
        scratch_shapes=[
            pltpu.VMEM((_G + _K, c), jnp.float32),
            pltpu.VMEM((_G, c), jnp.float32),
            pltpu.VMEM((_BRA, c), jnp.int32),
        ],
        compiler_params=pltpu.CompilerParams(
            dimension_semantics=("arbitrary",),
        ),
    )(bloa, bhia, x, bcola, browa, W1, W2)

    ypad = jnp.concatenate([y, jnp.zeros((_K, c), jnp.float32)], axis=0)

    nbb = n // _BRB
    bcolb = batch.reshape(nbb, _BRB, 1)
    blob = bcolb[:, 0, 0]
    bhib = bcolb[:, _BRB - 1, 0]

    out = pl.pallas_call(
        _pass_b,
        grid=(nbb,),
        in_specs=[
            pl.BlockSpec(memory_space=pltpu.SMEM),
            pl.BlockSpec(memory_space=pltpu.SMEM),
            pl.BlockSpec((_BRB, c), lambda i: (i, 0)),
            pl.BlockSpec((1, _BRB, 1), lambda i: (i, 0, 0)),
            pl.BlockSpec((_G + _K, c), lambda i: (0, 0)),
        ],
        out_specs=pl.BlockSpec((_BRB, c), lambda i: (i, 0)),
        out_shape=jax.ShapeDtypeStruct((n, c), jnp.float32),
        compiler_params=pltpu.CompilerParams(
            dimension_semantics=("arbitrary",),
        ),
    )(blob, bhib, x, bcolb, ypad)
    return out
